# baseline (speedup 1.0000x reference)
import functools

import jax
import jax.numpy as jnp
from jax import lax
from jax.experimental import pallas as pl
from jax.experimental.pallas import tpu as pltpu
from jax.experimental.pallas import tpu_sc as plsc

B = 16384
S = 200
D = 16
C = 8

NC = 2
NS = 16
NW = NC * NS
EPW = B // NW
CH = 128
NCHUNK = EPW // CH


def _sc_pool_body(xt_hbm, table_hbm, pooled_hbm, idx_v, g0, g1, acc, sem0, sem1):
    wid = lax.axis_index("s") * NC + lax.axis_index("c")
    base_elem = wid * EPW
    scale = jnp.float32(1.0 / S)

    def chunk_body(c, _):
        cb = base_elem + c * CH
        pltpu.sync_copy(xt_hbm.at[:, pl.ds(cb, CH)], idx_v)

        def fire(l, gbuf, sem):
            return pltpu.async_copy(table_hbm.at[idx_v.at[l]], gbuf, sem)

        def accum(gbuf, first):
            for k in range(CH):
                if first:
                    acc[k] = gbuf[k]
                else:
                    plsc.addupdate(acc.at[k], gbuf[k])

        fire(0, g0, sem0).wait()
        cp1 = fire(1, g1, sem1)
        accum(g0, True)
        cp1.wait()

        def pair_body(i, _):
            l = 2 * i
            cp0 = fire(l + 2, g0, sem0)
            accum(g1, False)
            cp0.wait()
            cp1 = fire(l + 3, g1, sem1)
            accum(g0, False)
            cp1.wait()
            return 0

        lax.fori_loop(0, (S - 2) // 2, pair_body, 0)
        accum(g1, False)

        for k in range(CH):
            acc[k] = acc[k] * scale
        pltpu.sync_copy(acc, pooled_hbm.at[pl.ds(cb, CH)])
        return 0

    lax.fori_loop(0, NCHUNK, chunk_body, 0)


@jax.jit
def _sc_pool(xt, table):
    mesh = plsc.VectorSubcoreMesh(core_axis_name="c", subcore_axis_name="s")
    return pl.kernel(
        _sc_pool_body,
        out_type=jax.ShapeDtypeStruct((B, D), jnp.float32),
        mesh=mesh,
        scratch_types=[
            pltpu.VMEM((S, CH), jnp.int32),
            pltpu.VMEM((CH, D), jnp.float32),
            pltpu.VMEM((CH, D), jnp.float32),
            pltpu.VMEM((CH, D), jnp.float32),
            pltpu.SemaphoreType.DMA,
            pltpu.SemaphoreType.DMA,
        ],
        compiler_params=pltpu.CompilerParams(use_tc_tiling_on_sc=False),
    )(xt, table)


def _tc_linear_body(p_ref, w_ref, b_ref, o_ref):
    o_ref[...] = (
        jnp.dot(p_ref[...], w_ref[...], preferred_element_type=jnp.float32)
        + b_ref[...]
    )


def kernel(x, table, W, b):
    pooled = _sc_pool(x.T, table)
    return pl.pallas_call(
        _tc_linear_body,
        out_shape=jax.ShapeDtypeStruct((B, C), jnp.float32),
    )(pooled, W, b.reshape(1, C))


# speedup vs baseline: 2.2321x; 2.2321x over previous
import jax
import jax.numpy as jnp
from jax import lax
from jax.experimental import pallas as pl
from jax.experimental.pallas import tpu as pltpu
from jax.experimental.pallas import tpu_sc as plsc

B = 16384
S = 200
D = 16
C = 8

NC = 2
NS = 16
NW = NC * NS
CH = 128
NG = B // CH
GPW = NG // NW
NBATCH = 8
BW = S // NBATCH


VB = 16384


def _t_relayout_body(tt_ref, o_ref):
    t = tt_ref[...].T.reshape(VB // 8, 8, D)
    o_ref[...] = jnp.concatenate([t[:, h, :] for h in range(8)], axis=1)


def _t_relayout(tt):
    v = tt.shape[1]
    return pl.pallas_call(
        _t_relayout_body,
        grid=(pl.cdiv(v, VB),),
        in_specs=[pl.BlockSpec((D, VB), lambda i: (0, i))],
        out_specs=pl.BlockSpec((VB // 8, 8 * D), lambda i: (i, 0)),
        out_shape=jax.ShapeDtypeStruct((v * D // 128, 128), jnp.float32),
    )(tt)


def _x_relayout_body(xt_ref, o_ref):
    o_ref[...] = xt_ref[...].reshape(1024, 128)


def _x_relayout(xt):
    return pl.pallas_call(
        _x_relayout_body,
        grid=(S // 8,),
        in_specs=[pl.BlockSpec((8, B), lambda i: (i, 0))],
        out_specs=pl.BlockSpec((1024, 128), lambda i: (i, 0)),
        out_shape=jax.ShapeDtypeStruct((S * B // 128, 128), jnp.int32),
    )(xt)


def _sc_pool_body(x3_hbm, table_hbm, pooled_hbm, idx_v, acc, sem0, sem1):
    wid = lax.axis_index("s") * NC + lax.axis_index("c")
    scale = jnp.float32(1.0 / S)
    sems = (sem0, sem1)
    zvec = jnp.zeros((D,), jnp.float32)

    def group_body(ci, _):
        g = wid * GPW + ci
        pltpu.sync_copy(x3_hbm.at[:, g, :], idx_v)
        for k in range(CH):
            acc[k] = zvec

        def fire_batch(bb):
            return [
                pltpu.async_copy(
                    table_hbm.at[idx_v.at[bb * BW + j]],
                    acc,
                    sems[bb % 2],
                    add=True,
                )
                for j in range(BW)
            ]

        pend = [fire_batch(0), fire_batch(1)]
        for bb in range(2, NBATCH):
            for cp in pend[bb % 2]:
                cp.wait()
            pend[bb % 2] = fire_batch(bb)
        for cp in pend[0]:
            cp.wait()
        for cp in pend[1]:
            cp.wait()

        for k in range(CH):
            acc[k] = acc[k] * scale
        pltpu.sync_copy(acc, pooled_hbm.at[pl.ds(g * CH, CH)])
        return 0

    lax.fori_loop(0, GPW, group_body, 0)


@jax.jit
def _sc_pool(x3, table):
    mesh = plsc.VectorSubcoreMesh(core_axis_name="c", subcore_axis_name="s")
    return pl.kernel(
        _sc_pool_body,
        out_type=jax.ShapeDtypeStruct((B, D), jnp.float32),
        mesh=mesh,
        scratch_types=[
            pltpu.VMEM((S, CH), jnp.int32),
            pltpu.VMEM((CH, D), jnp.float32),
            pltpu.SemaphoreType.DMA,
            pltpu.SemaphoreType.DMA,
        ],
        compiler_params=pltpu.CompilerParams(use_tc_tiling_on_sc=False),
    )(x3, table)


def _tc_linear_body(p_ref, w_ref, b_ref, o_ref):
    o_ref[...] = (
        jnp.dot(p_ref[...], w_ref[...], preferred_element_type=jnp.float32)
        + b_ref[...]
    )


def kernel(x, table, W, b):
    xlin = _x_relayout(x.T)
    x3 = xlin.reshape(S, B // CH, CH)
    table_lin = _t_relayout(table.T).reshape(table.shape)
    pooled = _sc_pool(x3, table_lin)
    return pl.pallas_call(
        _tc_linear_body,
        out_shape=jax.ShapeDtypeStruct((B, C), jnp.float32),
    )(pooled, W, b.reshape(1, C))


# speedup vs baseline: 4.2974x; 1.9253x over previous
import jax
import jax.numpy as jnp
from jax import lax
from jax.experimental import pallas as pl
from jax.experimental.pallas import tpu as pltpu
from jax.experimental.pallas import tpu_sc as plsc

B = 16384
S = 200
D = 16
C = 8

NC = 2
NS = 16
NW = NC * NS
CH = 128
NG = B // CH
GPW = NG // NW
NBATCH = 8
BW = S // NBATCH


VB = 16384
VOCAB = 1000000
NPAD = 1000448


def _t_relayout_body(tt_ref, o_ref):
    blk = tt_ref[...]
    s = blk.reshape(D, VB // 1024, 8, 128)
    s2 = jnp.transpose(s, (1, 2, 0, 3))
    s3 = s2.reshape(VB // 1024, 128, 128)
    s4 = jnp.transpose(s3, (0, 2, 1))
    o_ref[...] = s4.reshape(VB // 8, 128)


def _t_relayout(tt):
    return pl.pallas_call(
        _t_relayout_body,
        grid=(pl.cdiv(VOCAB, VB),),
        in_specs=[pl.BlockSpec((D, VB), lambda i: (0, i))],
        out_specs=pl.BlockSpec((VB // 8, 128), lambda i: (i, 0)),
        out_shape=jax.ShapeDtypeStruct((NPAD * D // 128, 128), jnp.float32),
    )(tt)


def _x_relayout_body(xt_ref, o_ref):
    v = xt_ref[...].reshape(1024, 128)
    o_ref[...] = (v & ~1023) | ((v & 127) << 3) | ((v >> 7) & 7)


def _x_relayout(xt):
    return pl.pallas_call(
        _x_relayout_body,
        grid=(S // 8,),
        in_specs=[pl.BlockSpec((8, B), lambda i: (i, 0))],
        out_specs=pl.BlockSpec((1024, 128), lambda i: (i, 0)),
        out_shape=jax.ShapeDtypeStruct((S * B // 128, 128), jnp.int32),
    )(xt)


def _sc_pool_body(x3_hbm, table_hbm, pooled_hbm, idx_v, acc, sem0, sem1):
    wid = lax.axis_index("s") * NC + lax.axis_index("c")
    scale = jnp.float32(1.0 / S)
    sems = (sem0, sem1)
    zvec = jnp.zeros((D,), jnp.float32)

    def group_body(ci, _):
        g = wid * GPW + ci
        pltpu.sync_copy(x3_hbm.at[:, g, :], idx_v)
        for k in range(CH):
            acc[k] = zvec

        def fire_batch(bb):
            return [
                pltpu.async_copy(
                    table_hbm.at[idx_v.at[bb * BW + j]],
                    acc,
                    sems[bb % 2],
                    add=True,
                )
                for j in range(BW)
            ]

        pend = [fire_batch(0), fire_batch(1)]
        for bb in range(2, NBATCH):
            for cp in pend[bb % 2]:
                cp.wait()
            pend[bb % 2] = fire_batch(bb)
        for cp in pend[0]:
            cp.wait()
        for cp in pend[1]:
            cp.wait()

        for k in range(CH):
            acc[k] = acc[k] * scale
        pltpu.sync_copy(acc, pooled_hbm.at[pl.ds(g * CH, CH)])
        return 0

    lax.fori_loop(0, GPW, group_body, 0)


@jax.jit
def _sc_pool(x3, table):
    mesh = plsc.VectorSubcoreMesh(core_axis_name="c", subcore_axis_name="s")
    return pl.kernel(
        _sc_pool_body,
        out_type=jax.ShapeDtypeStruct((B, D), jnp.float32),
        mesh=mesh,
        scratch_types=[
            pltpu.VMEM((S, CH), jnp.int32),
            pltpu.VMEM((CH, D), jnp.float32),
            pltpu.SemaphoreType.DMA,
            pltpu.SemaphoreType.DMA,
        ],
        compiler_params=pltpu.CompilerParams(use_tc_tiling_on_sc=False),
    )(x3, table)


def _tc_linear_body(p_ref, w_ref, b_ref, o_ref):
    o_ref[...] = (
        jnp.dot(p_ref[...], w_ref[...], preferred_element_type=jnp.float32)
        + b_ref[...]
    )


def kernel(x, table, W, b):
    xlin = _x_relayout(x.T)
    x3 = xlin.reshape(S, B // CH, CH)
    table_lin = _t_relayout(table.T).reshape(NPAD, D)
    pooled = _sc_pool(x3, table_lin)
    return pl.pallas_call(
        _tc_linear_body,
        out_shape=jax.ShapeDtypeStruct((B, C), jnp.float32),
    )(pooled, W, b.reshape(1, C))


# speedup vs baseline: 4.4184x; 1.0282x over previous
import jax
import jax.numpy as jnp
from jax import lax
from jax.experimental import pallas as pl
from jax.experimental.pallas import tpu as pltpu
from jax.experimental.pallas import tpu_sc as plsc

B = 16384
S = 200
D = 16
C = 8
VOCAB = 1000000

NC = 2
NS = 16
NW = NC * NS
CH = 128
NG = B // CH
GPW = NG // NW
NBATCH = 8
BW = S // NBATCH

VB = 16384
NOUT = 62 * (VB // 16)


def _proj_pack_body(w_ref, t_ref, b_ref, o_ref):
    blk = t_ref[...]
    pj = (jnp.dot(w_ref[...], blk, preferred_element_type=jnp.float32)
          + b_ref[...]) * jnp.float32(1.0 / S)
    s = pj.reshape(C, VB // 2048, 16, 128)
    s2 = jnp.transpose(s, (1, 2, 0, 3))
    s3 = s2.reshape(VB // 2048, 128, 128)
    s4 = jnp.transpose(s3, (0, 2, 1))
    o_ref[...] = s4.reshape(VB // 16, 128)


def _proj_pack(wT, tt, b2):
    return pl.pallas_call(
        _proj_pack_body,
        grid=(pl.cdiv(VOCAB, VB),),
        in_specs=[
            pl.BlockSpec((C, D), lambda i: (0, 0)),
            pl.BlockSpec((D, VB), lambda i: (0, i)),
            pl.BlockSpec((C, 1), lambda i: (0, 0)),
        ],
        out_specs=pl.BlockSpec((VB // 16, 128), lambda i: (i, 0)),
        out_shape=jax.ShapeDtypeStruct((NOUT, 128), jnp.float32),
    )(wT, tt, b2)


def _x_relayout_body(xt_ref, o_ref):
    v = xt_ref[...].reshape(1024, 128)
    o_ref[...] = (v & ~2047) | ((v & 127) << 4) | ((v >> 7) & 15)


def _x_relayout(xt):
    return pl.pallas_call(
        _x_relayout_body,
        grid=(S // 8,),
        in_specs=[pl.BlockSpec((8, B), lambda i: (i, 0))],
        out_specs=pl.BlockSpec((1024, 128), lambda i: (i, 0)),
        out_shape=jax.ShapeDtypeStruct((S * B // 128, 128), jnp.int32),
    )(xt)


def _sc_pool_body(x3_hbm, proj_hbm, zeros_hbm, out_hbm, idx_v, acc, sem0, sem1):
    wid = lax.axis_index("s") * NC + lax.axis_index("c")
    sems = (sem0, sem1)

    def group_body(ci, _):
        g = wid * GPW + ci
        pltpu.sync_copy(x3_hbm.at[:, g, :], idx_v)
        pltpu.sync_copy(zeros_hbm, acc)

        def fire_batch(bb):
            return [
                pltpu.async_copy(
                    proj_hbm.at[idx_v.at[bb * BW + j]],
                    acc,
                    sems[bb % 2],
                    add=True,
                )
                for j in range(BW)
            ]

        pend = [fire_batch(0), fire_batch(1)]
        for bb in range(2, NBATCH):
            for cp in pend[bb % 2]:
                cp.wait()
            pend[bb % 2] = fire_batch(bb)
        for cp in pend[0]:
            cp.wait()
        for cp in pend[1]:
            cp.wait()

        pltpu.sync_copy(acc, out_hbm.at[pl.ds(g * CH, CH)])
        return 0

    lax.fori_loop(0, GPW, group_body, 0)


@jax.jit
def _sc_pool(x3, proj, zeros):
    mesh = plsc.VectorSubcoreMesh(core_axis_name="c", subcore_axis_name="s")
    return pl.kernel(
        _sc_pool_body,
        out_type=jax.ShapeDtypeStruct((B, C), jnp.float32),
        mesh=mesh,
        scratch_types=[
            pltpu.VMEM((S, CH), jnp.int32),
            pltpu.VMEM((CH, C), jnp.float32),
            pltpu.SemaphoreType.DMA,
            pltpu.SemaphoreType.DMA,
        ],
        compiler_params=pltpu.CompilerParams(use_tc_tiling_on_sc=False),
    )(x3, proj, zeros)


def kernel(x, table, W, b):
    x3 = _x_relayout(x.T).reshape(S, B // CH, CH)
    proj = _proj_pack(W.T, table.T, b.reshape(C, 1)).reshape(NOUT * 16, C)
    zeros = jnp.zeros((CH, C), jnp.float32)
    return _sc_pool(x3, proj, zeros)


# speedup vs baseline: 5.0042x; 1.1326x over previous
import jax
import jax.numpy as jnp
from jax import lax
from jax.experimental import pallas as pl
from jax.experimental.pallas import tpu as pltpu
from jax.experimental.pallas import tpu_sc as plsc

B = 16384
S = 200
D = 16
C = 8
VOCAB = 1000000

NC = 2
NS = 16
NW = NC * NS
CH = 128
NG = B // CH
GPW = NG // NW
NBATCH = 8
BW = S // NBATCH

VB = 32768
NOUT = 31 * (VB // 16)


def _proj_pack_body(w_ref, t_ref, b_ref, o_ref):
    blk = t_ref[...]
    pj = (jnp.dot(w_ref[...], blk, preferred_element_type=jnp.float32)
          + b_ref[...]) * jnp.float32(1.0 / S)
    s = pj.reshape(C, VB // 2048, 16, 128)
    s2 = jnp.transpose(s, (1, 2, 0, 3))
    s3 = s2.reshape(VB // 2048, 128, 128)
    s4 = jnp.transpose(s3, (0, 2, 1))
    o_ref[...] = s4.reshape(VB // 16, 128)


def _proj_pack(wT, tt, b2):
    return pl.pallas_call(
        _proj_pack_body,
        grid=(pl.cdiv(VOCAB, VB),),
        in_specs=[
            pl.BlockSpec((C, D), lambda i: (0, 0)),
            pl.BlockSpec((D, VB), lambda i: (0, i)),
            pl.BlockSpec((C, 1), lambda i: (0, 0)),
        ],
        out_specs=pl.BlockSpec((VB // 16, 128), lambda i: (i, 0)),
        out_shape=jax.ShapeDtypeStruct((NOUT, 128), jnp.float32),
    )(wT, tt, b2)


def _x_relayout_body(xt_ref, o_ref):
    v = xt_ref[...].reshape(1024, 128)
    o_ref[...] = (v & ~2047) | ((v & 127) << 4) | ((v >> 7) & 15)


def _x_relayout(xt):
    return pl.pallas_call(
        _x_relayout_body,
        grid=(S // 8,),
        in_specs=[pl.BlockSpec((8, B), lambda i: (i, 0))],
        out_specs=pl.BlockSpec((1024, 128), lambda i: (i, 0)),
        out_shape=jax.ShapeDtypeStruct((S * B // 128, 128), jnp.int32),
    )(xt)


def _sc_pool_body(x3_hbm, proj_hbm, zeros_hbm, out_hbm, idx_v, acc, accT, sem0, sem1):
    wid = lax.axis_index("s") * NC + lax.axis_index("c")
    sems = (sem0, sem1)
    lane = lax.iota(jnp.int32, 16)

    def group_body(ci, _):
        g = wid * GPW + ci
        pltpu.sync_copy(x3_hbm.at[:, g, :], idx_v)
        pltpu.sync_copy(zeros_hbm, acc)

        def fire_batch(bb):
            return [
                pltpu.async_copy(
                    proj_hbm.at[idx_v.at[bb * BW + j]],
                    acc,
                    sems[bb % 2],
                    add=True,
                )
                for j in range(BW)
            ]

        pend = [fire_batch(0), fire_batch(1)]
        for bb in range(2, NBATCH):
            for cp in pend[bb % 2]:
                cp.wait()
            pend[bb % 2] = fire_batch(bb)
        for cp in pend[0]:
            cp.wait()
        for cp in pend[1]:
            cp.wait()

        for k in range(C):
            kvec = jnp.full((16,), k, jnp.int32)
            for j in range(CH // 16):
                accT[k, pl.ds(16 * j, 16)] = plsc.load_gather(
                    acc, [lane + 16 * j, kvec]
                )
        pltpu.sync_copy(accT, out_hbm.at[:, pl.ds(g * CH, CH)])
        return 0

    lax.fori_loop(0, GPW, group_body, 0)


@jax.jit
def _sc_pool(x3, proj, zeros):
    mesh = plsc.VectorSubcoreMesh(core_axis_name="c", subcore_axis_name="s")
    return pl.kernel(
        _sc_pool_body,
        out_type=jax.ShapeDtypeStruct((C, B), jnp.float32),
        mesh=mesh,
        scratch_types=[
            pltpu.VMEM((S, CH), jnp.int32),
            pltpu.VMEM((CH, C), jnp.float32),
            pltpu.VMEM((C, CH), jnp.float32),
            pltpu.SemaphoreType.DMA,
            pltpu.SemaphoreType.DMA,
        ],
        compiler_params=pltpu.CompilerParams(
            use_tc_tiling_on_sc=False, needs_layout_passes=False
        ),
    )(x3, proj, zeros)


def kernel(x, table, W, b):
    x3 = _x_relayout(x.T).reshape(S, B // CH, CH)
    proj = _proj_pack(W.T, table.T, b.reshape(C, 1)).reshape(NOUT * 16, C)
    zeros = jnp.zeros((CH, C), jnp.float32)
    return _sc_pool(x3, proj, zeros).T


# speedup vs baseline: 5.4432x; 1.0877x over previous
import jax
import jax.numpy as jnp
from jax import lax
from jax.experimental import pallas as pl
from jax.experimental.pallas import tpu as pltpu
from jax.experimental.pallas import tpu_sc as plsc

B = 16384
S = 200
D = 16
C = 8
VOCAB = 1000000

NC = 2
NS = 16
NW = NC * NS
CH = 128
NG = B // CH
GPW = NG // NW
NBATCH = 8
BW = S // NBATCH

VB = 65536
NOUT = 16 * (VB // 16)


def _proj_pack_body(w_ref, t_ref, b_ref, o_ref):
    blk = t_ref[...]
    pj = (jnp.dot(w_ref[...], blk, preferred_element_type=jnp.float32)
          + b_ref[...]) * jnp.float32(1.0 / S)
    s = pj.reshape(C, VB // 2048, 16, 128)
    s2 = jnp.transpose(s, (1, 2, 0, 3))
    s3 = s2.reshape(VB // 2048, 128, 128)
    s4 = jnp.transpose(s3, (0, 2, 1))
    o_ref[...] = s4.reshape(VB // 16, 128)


def _proj_pack(wT, tt, b2):
    return pl.pallas_call(
        _proj_pack_body,
        grid=(pl.cdiv(VOCAB, VB),),
        in_specs=[
            pl.BlockSpec((C, D), lambda i: (0, 0)),
            pl.BlockSpec((D, VB), lambda i: (0, i)),
            pl.BlockSpec((C, 1), lambda i: (0, 0)),
        ],
        out_specs=pl.BlockSpec((VB // 16, 128), lambda i: (i, 0)),
        out_shape=jax.ShapeDtypeStruct((NOUT, 128), jnp.float32),
    )(wT, tt, b2)


XROWS = 40


def _x_relayout_body(xt_ref, o_ref):
    v = xt_ref[...].reshape(XROWS * 128, 128)
    o_ref[...] = (v & ~2047) | ((v & 127) << 4) | ((v >> 7) & 15)


def _x_relayout(xt):
    return pl.pallas_call(
        _x_relayout_body,
        grid=(S // XROWS,),
        in_specs=[pl.BlockSpec((XROWS, B), lambda i: (i, 0))],
        out_specs=pl.BlockSpec((XROWS * 128, 128), lambda i: (i, 0)),
        out_shape=jax.ShapeDtypeStruct((S * B // 128, 128), jnp.int32),
    )(xt)


def _sc_pool_body(x3_hbm, proj_hbm, zeros_hbm, out_hbm, idx_v, acc, accT, sem0, sem1):
    wid = lax.axis_index("s") * NC + lax.axis_index("c")
    sems = (sem0, sem1)
    lane = lax.iota(jnp.int32, 16)

    def group_body(ci, _):
        g = wid * GPW + ci
        pltpu.sync_copy(x3_hbm.at[:, g, :], idx_v)
        pltpu.sync_copy(zeros_hbm, acc)

        def fire_batch(bb):
            return [
                pltpu.async_copy(
                    proj_hbm.at[idx_v.at[bb * BW + j]],
                    acc,
                    sems[bb % 2],
                    add=True,
                )
                for j in range(BW)
            ]

        pend = [fire_batch(0), fire_batch(1)]
        for bb in range(2, NBATCH):
            for cp in pend[bb % 2]:
                cp.wait()
            pend[bb % 2] = fire_batch(bb)
        for cp in pend[0]:
            cp.wait()
        for cp in pend[1]:
            cp.wait()

        for k in range(C):
            kvec = jnp.full((16,), k, jnp.int32)
            for j in range(CH // 16):
                accT[k, pl.ds(16 * j, 16)] = plsc.load_gather(
                    acc, [lane + 16 * j, kvec]
                )
        pltpu.sync_copy(accT, out_hbm.at[:, pl.ds(g * CH, CH)])
        return 0

    lax.fori_loop(0, GPW, group_body, 0)


@jax.jit
def _sc_pool(x3, proj, zeros):
    mesh = plsc.VectorSubcoreMesh(core_axis_name="c", subcore_axis_name="s")
    return pl.kernel(
        _sc_pool_body,
        out_type=jax.ShapeDtypeStruct((C, B), jnp.float32),
        mesh=mesh,
        scratch_types=[
            pltpu.VMEM((S, CH), jnp.int32),
            pltpu.VMEM((CH, C), jnp.float32),
            pltpu.VMEM((C, CH), jnp.float32),
            pltpu.SemaphoreType.DMA,
            pltpu.SemaphoreType.DMA,
        ],
        compiler_params=pltpu.CompilerParams(
            use_tc_tiling_on_sc=False, needs_layout_passes=False
        ),
    )(x3, proj, zeros)


def kernel(x, table, W, b):
    x3 = _x_relayout(x.T).reshape(S, B // CH, CH)
    proj = _proj_pack(W.T, table.T, b.reshape(C, 1)).reshape(NOUT * 16, C)
    zeros = jnp.zeros((CH, C), jnp.float32)
    return _sc_pool(x3, proj, zeros).T
